# baseline (speedup 1.0000x reference)
import jax
import jax.numpy as jnp
from jax.experimental import pallas as pl
from jax.experimental.pallas import tpu as pltpu

N_NODES = 2048
N_HYPER = 32
_SLICES = 2
D_MODEL = 128
ALPHA = 1.0
BETA = 0.5
GAMMA = 0.5
NEG_SLOPE = 0.02

_PREC = jax.lax.Precision.DEFAULT


def _mm(a, b):
    return jax.lax.dot_general(a, b, (((1,), (0,)), ((), ())),
                               precision=_PREC, preferred_element_type=jnp.float32)


def _mm_t(a, b):
    return jax.lax.dot_general(a, b, (((0,), (0,)), ((), ())),
                               precision=_PREC, preferred_element_type=jnp.float32)


def _mm_rt(a, b):
    return jax.lax.dot_general(a, b, (((1,), (1,)), ((), ())),
                               precision=_PREC, preferred_element_type=jnp.float32)


def _hyper_body(x_ref, en_ref, eh_ref, w_ref, bias_ref, a_ref,
                out_ref, loss_ref,
                A_s, AT_s, NM_s, degnr_s, degec_s, deger_s, binvc_s, binvr_s,
                WT_s, biasc_s, a1r_s, a2r_s, acclnr_s, accM_s):
    bidx = pl.program_id(0)
    nb = pl.num_programs(0)

    @pl.when(bidx == 0)
    def _init():
        aff = _mm_rt(en_ref[...], eh_ref[...])
        z = jnp.maximum(ALPHA * aff, 0.0)
        zmax = jnp.max(z, axis=1, keepdims=True)
        ez = jnp.exp(z - zmax)
        p = ez / jnp.sum(ez, axis=1, keepdims=True)
        A = (p > BETA).astype(jnp.float32)
        A_s[...] = A
        AT = jnp.transpose(A)
        AT_s[...] = AT
        NM_s[...] = jnp.where(AT > 0, 0.0, -jnp.inf)
        degnr_s[...] = jnp.sum(AT, axis=0, keepdims=True)
        dege_col = jnp.sum(AT, axis=1, keepdims=True)
        degec_s[...] = dege_col
        deger_s[...] = jnp.transpose(dege_col)
        binv_col = jnp.where(dege_col > 0, 1.0 / dege_col, 0.0)
        binvc_s[...] = binv_col
        binvr_s[...] = jnp.transpose(binv_col)
        WT_s[...] = jnp.transpose(w_ref[...])
        biasc_s[...] = jnp.transpose(bias_ref[...])
        a1r_s[...] = jnp.transpose(a_ref[0:D_MODEL, :])
        a2r_s[...] = jnp.transpose(a_ref[D_MODEL:2 * D_MODEL, :])
        acclnr_s[...] = jnp.zeros((1, N_NODES), jnp.float32)
        accM_s[...] = jnp.zeros((N_HYPER, N_HYPER), jnp.float32)

    A = A_s[...]
    AT = AT_s[...]

    ns = x_ref.shape[0]
    xwTs, resTs, lgs, colsums = [], [], [], []
    for j in range(ns):
        xbT = jnp.transpose(x_ref[j])
        xwT = _mm(WT_s[...], xbT) + biasc_s[...]
        segT = _mm(xwT, A)
        resT = segT * binvr_s[...]
        GT = _mm(resT, AT)
        colsums.append(jnp.sum(jnp.abs(xwT - GT), axis=0, keepdims=True))
        s1_row = _mm(a1r_s[...], xwT)
        s2_row = _mm(a2r_s[...], resT)
        s2g_row = _mm(s2_row, AT)
        lg = s1_row + s2g_row
        lgs.append(jnp.where(lg >= 0, lg, NEG_SLOPE * lg))
        xwTs.append(xwT)
        resTs.append(resT)

    acclnr_s[...] = acclnr_s[...] + degnr_s[...] * sum(colsums)

    LG = jnp.concatenate(lgs, axis=0)[None, :, :]
    masked = NM_s[...][:, None, :] + LG
    m3 = jnp.max(masked, axis=2, keepdims=True)
    m3 = jnp.where(degec_s[...][:, None, :] > 0, m3, 0.0)
    E3 = jnp.exp(masked - m3)
    s3 = jnp.sum(E3, axis=2, keepdims=True)
    r3 = 1.0 / (s3 + 1e-16)

    res2 = []
    for j in range(ns):
        out1T = _mm(xwTs[j], jnp.transpose(E3[:, j, :]))
        out1T = out1T * jnp.transpose(r3[:, j, :])
        res2.append(binvc_s[...] * jnp.transpose(out1T))
    out_all = _mm(A, jnp.concatenate(res2, axis=1))
    for j in range(ns):
        out_ref[j] = out_all[:, j * D_MODEL:(j + 1) * D_MODEL]

    Gms, sqs = [], []
    for j in range(ns):
        Gms.append(_mm_t(resTs[j], resTs[j])[None])
        sqs.append(jnp.sum(resTs[j] * resTs[j], axis=0, keepdims=True)[None])
    GM = jnp.concatenate(Gms, axis=0)
    SQR = jnp.concatenate(sqs, axis=0)
    SQC = jnp.transpose(SQR, (0, 2, 1))
    d2 = jnp.maximum(SQC + SQR - 2.0 * GM, 0.0)
    dist = jnp.sqrt(d2 + 1e-8)
    nrm_r = jnp.maximum(jnp.sqrt(SQR), 1e-12)
    nrm_c = jnp.maximum(jnp.sqrt(SQC), 1e-12)
    cos = GM / (nrm_c * nrm_r)
    items = cos * dist + (1.0 - cos) * jnp.maximum(GAMMA - dist, 0.0)
    accM_s[...] = accM_s[...] + jnp.sum(items, axis=0)

    @pl.when(bidx == nb - 1)
    def _fin():
        dege = deger_s[...]
        P = jnp.sum(dege)
        keep_row = (dege > 0).astype(jnp.float32)
        numE = jnp.sum(keep_row)
        ln = jnp.sum(acclnr_s[...]) / (P * (nb * x_ref.shape[0]) * D_MODEL)
        Mmean = accM_s[...] / (nb * x_ref.shape[0])
        keep_col = (degec_s[...] > 0).astype(jnp.float32)
        pair = keep_col * keep_row
        lh = jnp.sum(jnp.abs(Mmean) * pair) / (numE * numE)
        loss_ref[...] = (ln + lh)[None, None]


def kernel(x, embed_hyper, embed_node, weight, bias, a, b):
    B, L, S, D = x.shape
    x2 = x.reshape(B * L, S, D)
    bias2 = bias.reshape(1, D)
    out, loss = pl.pallas_call(
        _hyper_body,
        grid=(B * L // _SLICES,),
        in_specs=[
            pl.BlockSpec((_SLICES, S, D), lambda i: (i, 0, 0)),
            pl.BlockSpec((S, D), lambda i: (0, 0)),
            pl.BlockSpec((N_HYPER, D), lambda i: (0, 0)),
            pl.BlockSpec((D, D), lambda i: (0, 0)),
            pl.BlockSpec((1, D), lambda i: (0, 0)),
            pl.BlockSpec((2 * D, 1), lambda i: (0, 0)),
        ],
        out_specs=[
            pl.BlockSpec((_SLICES, S, D), lambda i: (i, 0, 0)),
            pl.BlockSpec((1, 1), lambda i: (0, 0)),
        ],
        out_shape=[
            jax.ShapeDtypeStruct((B * L, S, D), jnp.float32),
            jax.ShapeDtypeStruct((1, 1), jnp.float32),
        ],
        scratch_shapes=[
            pltpu.VMEM((N_NODES, N_HYPER), jnp.float32),
            pltpu.VMEM((N_HYPER, N_NODES), jnp.float32),
            pltpu.VMEM((N_HYPER, N_NODES), jnp.float32),
            pltpu.VMEM((1, N_NODES), jnp.float32),
            pltpu.VMEM((N_HYPER, 1), jnp.float32),
            pltpu.VMEM((1, N_HYPER), jnp.float32),
            pltpu.VMEM((N_HYPER, 1), jnp.float32),
            pltpu.VMEM((1, N_HYPER), jnp.float32),
            pltpu.VMEM((D_MODEL, D_MODEL), jnp.float32),
            pltpu.VMEM((D_MODEL, 1), jnp.float32),
            pltpu.VMEM((1, D_MODEL), jnp.float32),
            pltpu.VMEM((1, D_MODEL), jnp.float32),
            pltpu.VMEM((1, N_NODES), jnp.float32),
            pltpu.VMEM((N_HYPER, N_HYPER), jnp.float32),
        ],
    )(x2, embed_node, embed_hyper, weight, bias2, a)
    return out.reshape(B, L, S, D), loss[0, 0]


# speedup vs baseline: 1.2333x; 1.2333x over previous
import jax
import jax.numpy as jnp
from jax.experimental import pallas as pl
from jax.experimental.pallas import tpu as pltpu

N_NODES = 2048
N_HYPER = 32
_SLICES = 4
D_MODEL = 128
ALPHA = 1.0
BETA = 0.5
GAMMA = 0.5
NEG_SLOPE = 0.02

_PREC = jax.lax.Precision.DEFAULT


def _mm(a, b):
    return jax.lax.dot_general(a, b, (((1,), (0,)), ((), ())),
                               precision=_PREC, preferred_element_type=jnp.float32)


def _mm_t(a, b):
    return jax.lax.dot_general(a, b, (((0,), (0,)), ((), ())),
                               precision=_PREC, preferred_element_type=jnp.float32)


def _mm_rt(a, b):
    return jax.lax.dot_general(a, b, (((1,), (1,)), ((), ())),
                               precision=_PREC, preferred_element_type=jnp.float32)


def _hyper_body(x_ref, en_ref, eh_ref, w_ref, bias_ref, a_ref,
                out_ref, loss_ref,
                A_s, AT_s, NM_s, degnr_s, degec_s, deger_s, binvc_s, binvr_s,
                WT_s, biasc_s, a1r_s, a2r_s, acclnr_s, accM_s):
    bidx = pl.program_id(0)
    nb = pl.num_programs(0)

    @pl.when(bidx == 0)
    def _init():
        aff = _mm_rt(en_ref[...], eh_ref[...])
        z = jnp.maximum(ALPHA * aff, 0.0)
        zmax = jnp.max(z, axis=1, keepdims=True)
        ez = jnp.exp(z - zmax)
        p = ez / jnp.sum(ez, axis=1, keepdims=True)
        A = (p > BETA).astype(jnp.float32)
        A_s[...] = A
        AT = jnp.transpose(A)
        AT_s[...] = AT
        NM_s[...] = jnp.where(AT > 0, 0.0, -jnp.inf)
        degnr_s[...] = jnp.sum(AT, axis=0, keepdims=True)
        dege_col = jnp.sum(AT, axis=1, keepdims=True)
        degec_s[...] = dege_col
        deger_s[...] = jnp.transpose(dege_col)
        binv_col = jnp.where(dege_col > 0, 1.0 / dege_col, 0.0)
        binvc_s[...] = binv_col
        binvr_s[...] = jnp.transpose(binv_col)
        WT_s[...] = jnp.transpose(w_ref[...])
        biasc_s[...] = jnp.transpose(bias_ref[...])
        a1r_s[...] = jnp.transpose(a_ref[0:D_MODEL, :])
        a2r_s[...] = jnp.transpose(a_ref[D_MODEL:2 * D_MODEL, :])
        acclnr_s[...] = jnp.zeros((1, N_NODES), jnp.float32)
        accM_s[...] = jnp.zeros((N_HYPER, N_HYPER), jnp.float32)

    A = A_s[...]
    AT = AT_s[...]

    ns = x_ref.shape[0]
    xwTs, resTs, lgs, colsums = [], [], [], []
    for j in range(ns):
        xwT = _mm_rt(WT_s[...], x_ref[j]) + biasc_s[...]
        segT = _mm(xwT, A)
        resT = segT * binvr_s[...]
        GT = _mm(resT, AT)
        colsums.append(jnp.sum(jnp.abs(xwT - GT), axis=0, keepdims=True))
        s1_row = _mm(a1r_s[...], xwT)
        s2_row = _mm(a2r_s[...], resT)
        s2g_row = _mm(s2_row, AT)
        lg = s1_row + s2g_row
        lgs.append(jnp.where(lg >= 0, lg, NEG_SLOPE * lg))
        xwTs.append(xwT)
        resTs.append(resT)

    acclnr_s[...] = acclnr_s[...] + degnr_s[...] * sum(colsums)

    LG = jnp.concatenate(lgs, axis=0)
    masked = NM_s[...][:, None, :] + LG[None, :, :]
    m3 = jnp.max(masked, axis=2)
    m3 = jnp.where(degec_s[...] > 0, m3, 0.0)
    m3g = _mm_t(m3, AT)
    garg = jnp.where(degnr_s[...] > 0, LG - m3g, -jnp.inf)
    G = jnp.exp(garg)
    s3 = _mm(G, A)
    r3 = 1.0 / (s3 + 1e-16)

    res2 = []
    for j in range(ns):
        out1T = _mm(xwTs[j] * G[j:j + 1, :], A)
        out1T = out1T * r3[j:j + 1, :]
        res2.append(binvc_s[...] * jnp.transpose(out1T))
    out_all = _mm(A, jnp.concatenate(res2, axis=1))
    for j in range(ns):
        out_ref[j] = out_all[:, j * D_MODEL:(j + 1) * D_MODEL]

    Gms, sqs = [], []
    for j in range(ns):
        Gms.append(_mm_t(resTs[j], resTs[j])[None])
        sqs.append(jnp.sum(resTs[j] * resTs[j], axis=0, keepdims=True)[None])
    GM = jnp.concatenate(Gms, axis=0)
    SQR = jnp.concatenate(sqs, axis=0)
    SQC = jnp.transpose(SQR, (0, 2, 1))
    d2 = jnp.maximum(SQC + SQR - 2.0 * GM, 0.0)
    dist = jnp.sqrt(d2 + 1e-8)
    nrm_r = jnp.maximum(jnp.sqrt(SQR), 1e-12)
    nrm_c = jnp.maximum(jnp.sqrt(SQC), 1e-12)
    cos = GM / (nrm_c * nrm_r)
    items = cos * dist + (1.0 - cos) * jnp.maximum(GAMMA - dist, 0.0)
    accM_s[...] = accM_s[...] + jnp.sum(items, axis=0)

    @pl.when(bidx == nb - 1)
    def _fin():
        dege = deger_s[...]
        P = jnp.sum(dege)
        keep_row = (dege > 0).astype(jnp.float32)
        numE = jnp.sum(keep_row)
        ln = jnp.sum(acclnr_s[...]) / (P * (nb * x_ref.shape[0]) * D_MODEL)
        Mmean = accM_s[...] / (nb * x_ref.shape[0])
        keep_col = (degec_s[...] > 0).astype(jnp.float32)
        pair = keep_col * keep_row
        lh = jnp.sum(jnp.abs(Mmean) * pair) / (numE * numE)
        loss_ref[...] = (ln + lh)[None, None]


def kernel(x, embed_hyper, embed_node, weight, bias, a, b):
    B, L, S, D = x.shape
    x2 = x.reshape(B * L, S, D)
    bias2 = bias.reshape(1, D)
    out, loss = pl.pallas_call(
        _hyper_body,
        grid=(B * L // _SLICES,),
        in_specs=[
            pl.BlockSpec((_SLICES, S, D), lambda i: (i, 0, 0)),
            pl.BlockSpec((S, D), lambda i: (0, 0)),
            pl.BlockSpec((N_HYPER, D), lambda i: (0, 0)),
            pl.BlockSpec((D, D), lambda i: (0, 0)),
            pl.BlockSpec((1, D), lambda i: (0, 0)),
            pl.BlockSpec((2 * D, 1), lambda i: (0, 0)),
        ],
        out_specs=[
            pl.BlockSpec((_SLICES, S, D), lambda i: (i, 0, 0)),
            pl.BlockSpec((1, 1), lambda i: (0, 0)),
        ],
        out_shape=[
            jax.ShapeDtypeStruct((B * L, S, D), jnp.float32),
            jax.ShapeDtypeStruct((1, 1), jnp.float32),
        ],
        scratch_shapes=[
            pltpu.VMEM((N_NODES, N_HYPER), jnp.float32),
            pltpu.VMEM((N_HYPER, N_NODES), jnp.float32),
            pltpu.VMEM((N_HYPER, N_NODES), jnp.float32),
            pltpu.VMEM((1, N_NODES), jnp.float32),
            pltpu.VMEM((N_HYPER, 1), jnp.float32),
            pltpu.VMEM((1, N_HYPER), jnp.float32),
            pltpu.VMEM((N_HYPER, 1), jnp.float32),
            pltpu.VMEM((1, N_HYPER), jnp.float32),
            pltpu.VMEM((D_MODEL, D_MODEL), jnp.float32),
            pltpu.VMEM((D_MODEL, 1), jnp.float32),
            pltpu.VMEM((1, D_MODEL), jnp.float32),
            pltpu.VMEM((1, D_MODEL), jnp.float32),
            pltpu.VMEM((1, N_NODES), jnp.float32),
            pltpu.VMEM((N_HYPER, N_HYPER), jnp.float32),
        ],
    )(x2, embed_node, embed_hyper, weight, bias2, a)
    return out.reshape(B, L, S, D), loss[0, 0]
